# baseline (speedup 1.0000x reference)
import functools

import jax
import jax.numpy as jnp
from jax import lax
from jax.experimental import pallas as pl
from jax.experimental.pallas import tpu as pltpu
from jax.experimental.pallas import tpu_sc as plsc

N_E = 8192
E_DIM = 256
BETA = 0.25
B_TOT = 16 * 576

TM = 256
TN = 512
N_ROW_TILES = B_TOT // TM
N_COL_CHUNKS = N_E // TN


def _argmin_body(rn_ref, en_ref, z_ref, e_ref, idx_ref, loss_ref):
    i = pl.program_id(0)
    z_tile = z_ref[...]
    rn_tile = rn_ref[...]

    def chunk(c, carry):
        m, ix = carry
        start = pl.multiple_of(c * TN, TN)
        e_chunk = e_ref[pl.ds(start, TN), :]
        en_chunk = en_ref[pl.ds(start, TN)]
        mm = lax.dot_general(z_tile, e_chunk, (((1,), (1,)), ((), ())),
                             preferred_element_type=jnp.float32)
        d = (rn_tile[:, None] + en_chunk[None, :]) - 2.0 * mm
        cmin = jnp.min(d, axis=1)
        iota = lax.broadcasted_iota(jnp.int32, (TM, TN), 1) + c * TN
        cidx = jnp.min(jnp.where(d == cmin[:, None], iota, jnp.int32(2**30)),
                       axis=1)
        better = cmin < m
        return jnp.where(better, cmin, m), jnp.where(better, cidx, ix)

    m0 = jnp.full((TM,), jnp.inf, jnp.float32)
    ix0 = jnp.zeros((TM,), jnp.int32)
    m, ix = lax.fori_loop(0, N_COL_CHUNKS, chunk, (m0, ix0))

    idx_ref[0, 0, :] = ix
    part = jnp.sum(m)

    @pl.when(i == 0)
    def _():
        loss_ref[0, 0] = part

    @pl.when(i > 0)
    def _():
        loss_ref[0, 0] += part

    @pl.when(i == N_ROW_TILES - 1)
    def _():
        loss_ref[0, 0] = loss_ref[0, 0] * ((1.0 + BETA) / (B_TOT * E_DIM))


_argmin_call = pl.pallas_call(
    _argmin_body,
    grid=(N_ROW_TILES,),
    in_specs=[
        pl.BlockSpec((TM,), lambda i: (i,)),
        pl.BlockSpec((N_E,), lambda i: (0,)),
        pl.BlockSpec((TM, E_DIM), lambda i: (i, 0)),
        pl.BlockSpec((N_E, E_DIM), lambda i: (0, 0)),
    ],
    out_specs=[
        pl.BlockSpec((1, 1, TM), lambda i: (i, 0, 0)),
        pl.BlockSpec(memory_space=pltpu.SMEM),
    ],
    out_shape=[
        jax.ShapeDtypeStruct((N_ROW_TILES, 1, TM), jnp.int32),
        jax.ShapeDtypeStruct((1, 1), jnp.float32),
    ],
)

_NW = 32
_RPW = B_TOT // _NW
_CH = 3
_CW = _RPW // _CH
_HSL = N_E // 16

_sc_mesh = plsc.VectorSubcoreMesh(core_axis_name="c", subcore_axis_name="s")


@functools.partial(
    pl.kernel,
    out_type=[
        jax.ShapeDtypeStruct((B_TOT, E_DIM), jnp.float32),
        jax.ShapeDtypeStruct((2, N_E), jnp.float32),
    ],
    mesh=_sc_mesh,
    scratch_types=[
        pltpu.VMEM((_CH, _CW), jnp.int32),
        pltpu.VMEM((_RPW, E_DIM), jnp.float32),
        pltpu.VMEM((_CW,), jnp.float32),
        pltpu.VMEM((_HSL,), jnp.float32),
        pltpu.VMEM_SHARED((N_E,), jnp.float32),
        pltpu.SemaphoreType.DMA,
    ],
)
def _sc_gather_hist(emb_hbm, idx_hbm, zq_hbm, cnt_hbm,
                    idx_v, rows_v, ones_v, zer_v, hist_sh, sem):
    c = lax.axis_index("c")
    s = lax.axis_index("s")
    wid = s * 2 + c
    pltpu.sync_copy(idx_hbm.at[wid], idx_v)

    handles = [
        pltpu.async_copy(emb_hbm.at[idx_v.at[j]],
                         rows_v.at[pl.ds(j * _CW, _CW)], sem)
        for j in range(_CH)
    ]

    for k in range(_HSL // 16):
        zer_v[pl.ds(k * 16, 16)] = jnp.zeros((16,), jnp.float32)
    for k in range(_CW // 16):
        ones_v[pl.ds(k * 16, 16)] = jnp.ones((16,), jnp.float32)
    pltpu.sync_copy(zer_v, hist_sh.at[pl.ds(s * _HSL, _HSL)])
    plsc.subcore_barrier()

    for j in range(_CH):
        pltpu.sync_copy(ones_v, hist_sh.at[idx_v.at[j]], add=True)
    plsc.subcore_barrier()

    @pl.when(s == 0)
    def _():
        pltpu.sync_copy(hist_sh, cnt_hbm.at[c])

    for h in handles:
        h.wait()
    pltpu.sync_copy(rows_v, zq_hbm.at[pl.ds(wid * _RPW, _RPW)])


def _perp_body(cnt_ref, out_ref):
    cnt = cnt_ref[...]
    tot = cnt[0:8, :] + cnt[8:16, :]
    avg = tot / jnp.float32(B_TOT)
    ent = jnp.sum(avg * jnp.log(avg + 1e-12))
    out_ref[0, 0] = jnp.exp(-ent)


_perp_call = pl.pallas_call(
    _perp_body,
    in_specs=[pl.BlockSpec((16, 1024), lambda: (0, 0))],
    out_specs=pl.BlockSpec(memory_space=pltpu.SMEM),
    out_shape=jax.ShapeDtypeStruct((1, 1), jnp.float32),
)


def kernel(z, embedding):
    bz = z.shape[0]
    z_flat = z.reshape(-1, E_DIM)
    rn = jnp.sum(z_flat ** 2, axis=1)
    en = jnp.sum(embedding ** 2, axis=1)

    idx_tiles, loss = _argmin_call(rn, en, z_flat, embedding)
    idx_flat = idx_tiles.reshape(B_TOT)

    zq_flat, counts = _sc_gather_hist(embedding,
                                      idx_flat.reshape(_NW, _CH, _CW))
    perp = _perp_call(counts.reshape(16, 1024))

    z_q = zq_flat.reshape(bz, -1, E_DIM)
    return (z_q, loss.reshape(()), idx_flat, perp.reshape(()))


# speedup vs baseline: 2.1317x; 2.1317x over previous
import functools

import jax
import jax.numpy as jnp
from jax import lax
from jax.experimental import pallas as pl
from jax.experimental.pallas import tpu as pltpu
from jax.experimental.pallas import tpu_sc as plsc

N_E = 8192
E_DIM = 256
BETA = 0.25
B_TOT = 16 * 576

TM = 256
TN = 512
N_ROW_TILES = B_TOT // TM
N_COL_CHUNKS = N_E // TN


def _argmin_body(rn_ref, en_ref, z_ref, e_ref, idx_ref, loss_ref):
    i = pl.program_id(0)
    z_tile = z_ref[...]
    rn_tile = rn_ref[...]

    m_t = jnp.full((TM, TN), jnp.inf, jnp.float32)
    cix_t = jnp.zeros((TM, TN), jnp.int32)
    for c in range(N_COL_CHUNKS):
        e_chunk = e_ref[pl.ds(c * TN, TN), :]
        en_chunk = en_ref[pl.ds(c * TN, TN)]
        mm = lax.dot_general(z_tile, e_chunk, (((1,), (1,)), ((), ())),
                             preferred_element_type=jnp.float32)
        d = (rn_tile[:, None] + en_chunk[None, :]) - 2.0 * mm
        upd = d < m_t
        m_t = jnp.where(upd, d, m_t)
        cix_t = jnp.where(upd, jnp.int32(c), cix_t)

    m = jnp.min(m_t, axis=1)
    giota = lax.broadcasted_iota(jnp.int32, (TM, TN), 1)
    gidx = cix_t * TN + giota
    ix = jnp.min(jnp.where(m_t == m[:, None], gidx, jnp.int32(2**30)), axis=1)

    idx_ref[0, 0, :] = ix
    part = jnp.sum(m)

    @pl.when(i == 0)
    def _():
        loss_ref[0, 0] = part

    @pl.when(i > 0)
    def _():
        loss_ref[0, 0] += part

    @pl.when(i == N_ROW_TILES - 1)
    def _():
        loss_ref[0, 0] = loss_ref[0, 0] * ((1.0 + BETA) / (B_TOT * E_DIM))


_argmin_call = pl.pallas_call(
    _argmin_body,
    grid=(N_ROW_TILES,),
    in_specs=[
        pl.BlockSpec((TM,), lambda i: (i,)),
        pl.BlockSpec((N_E,), lambda i: (0,)),
        pl.BlockSpec((TM, E_DIM), lambda i: (i, 0)),
        pl.BlockSpec((N_E, E_DIM), lambda i: (0, 0)),
    ],
    out_specs=[
        pl.BlockSpec((1, 1, TM), lambda i: (i, 0, 0)),
        pl.BlockSpec(memory_space=pltpu.SMEM),
    ],
    out_shape=[
        jax.ShapeDtypeStruct((N_ROW_TILES, 1, TM), jnp.int32),
        jax.ShapeDtypeStruct((1, 1), jnp.float32),
    ],
)

_NW = 32
_RPW = B_TOT // _NW
_CH = 3
_CW = _RPW // _CH
_HSL = N_E // 16

_sc_mesh = plsc.VectorSubcoreMesh(core_axis_name="c", subcore_axis_name="s")


@functools.partial(
    pl.kernel,
    out_type=[
        jax.ShapeDtypeStruct((B_TOT, E_DIM), jnp.float32),
        jax.ShapeDtypeStruct((2, N_E), jnp.float32),
    ],
    mesh=_sc_mesh,
    scratch_types=[
        pltpu.VMEM((_CH, _CW), jnp.int32),
        pltpu.VMEM((_RPW, E_DIM), jnp.float32),
        pltpu.VMEM((_CW,), jnp.float32),
        pltpu.VMEM((_HSL,), jnp.float32),
        pltpu.VMEM_SHARED((N_E,), jnp.float32),
        pltpu.SemaphoreType.DMA,
    ],
)
def _sc_gather_hist(emb_hbm, idx_hbm, zq_hbm, cnt_hbm,
                    idx_v, rows_v, ones_v, zer_v, hist_sh, sem):
    c = lax.axis_index("c")
    s = lax.axis_index("s")
    wid = s * 2 + c
    pltpu.sync_copy(idx_hbm.at[wid], idx_v)

    handles = [
        pltpu.async_copy(emb_hbm.at[idx_v.at[j]],
                         rows_v.at[pl.ds(j * _CW, _CW)], sem)
        for j in range(_CH)
    ]

    for k in range(_HSL // 16):
        zer_v[pl.ds(k * 16, 16)] = jnp.zeros((16,), jnp.float32)
    for k in range(_CW // 16):
        ones_v[pl.ds(k * 16, 16)] = jnp.ones((16,), jnp.float32)
    pltpu.sync_copy(zer_v, hist_sh.at[pl.ds(s * _HSL, _HSL)])
    plsc.subcore_barrier()

    for j in range(_CH):
        pltpu.sync_copy(ones_v, hist_sh.at[idx_v.at[j]], add=True)
    plsc.subcore_barrier()

    @pl.when(s == 0)
    def _():
        pltpu.sync_copy(hist_sh, cnt_hbm.at[c])

    for h in handles:
        h.wait()
    pltpu.sync_copy(rows_v, zq_hbm.at[pl.ds(wid * _RPW, _RPW)])


def _perp_body(cnt_ref, out_ref):
    cnt = cnt_ref[...]
    tot = cnt[0:8, :] + cnt[8:16, :]
    avg = tot / jnp.float32(B_TOT)
    ent = jnp.sum(avg * jnp.log(avg + 1e-12))
    out_ref[0, 0] = jnp.exp(-ent)


_perp_call = pl.pallas_call(
    _perp_body,
    in_specs=[pl.BlockSpec((16, 1024), lambda: (0, 0))],
    out_specs=pl.BlockSpec(memory_space=pltpu.SMEM),
    out_shape=jax.ShapeDtypeStruct((1, 1), jnp.float32),
)


def kernel(z, embedding):
    bz = z.shape[0]
    z_flat = z.reshape(-1, E_DIM)
    rn = jnp.sum(z_flat ** 2, axis=1)
    en = jnp.sum(embedding ** 2, axis=1)

    idx_tiles, loss = _argmin_call(rn, en, z_flat, embedding)
    idx_flat = idx_tiles.reshape(B_TOT)

    zq_flat, counts = _sc_gather_hist(embedding,
                                      idx_flat.reshape(_NW, _CH, _CW))
    perp = _perp_call(counts.reshape(16, 1024))

    z_q = zq_flat.reshape(bz, -1, E_DIM)
    return (z_q, loss.reshape(()), idx_flat, perp.reshape(()))


# speedup vs baseline: 2.2949x; 1.0766x over previous
import functools

import jax
import jax.numpy as jnp
from jax import lax
from jax.experimental import pallas as pl
from jax.experimental.pallas import tpu as pltpu
from jax.experimental.pallas import tpu_sc as plsc

N_E = 8192
E_DIM = 256
BETA = 0.25
B_TOT = 16 * 576

TM = 256
TN = 512
N_ROW_TILES = B_TOT // TM
N_COL_CHUNKS = N_E // TN


def _argmin_body(rn_ref, en_ref, z_ref, e_ref, idx_ref, loss_ref):
    i = pl.program_id(0)
    z_tile = z_ref[...]
    rn_tile = rn_ref[...]

    m_t = jnp.full((TM, TN), jnp.inf, jnp.float32)
    cix_t = jnp.zeros((TM, TN), jnp.int32)
    for p in range(N_COL_CHUNKS // 4):
        dd = []
        for q in range(4):
            c = p * 4 + q
            e_chunk = e_ref[pl.ds(c * TN, TN), :]
            en_chunk = en_ref[pl.ds(c * TN, TN)]
            mm = lax.dot_general(z_tile, e_chunk, (((1,), (1,)), ((), ())),
                                 preferred_element_type=jnp.float32)
            dd.append((rn_tile[:, None] + en_chunk[None, :]) - 2.0 * mm)
        u01 = dd[1] < dd[0]
        m01 = jnp.where(u01, dd[1], dd[0])
        c01 = jnp.where(u01, jnp.int32(1), jnp.int32(0))
        u23 = dd[3] < dd[2]
        m23 = jnp.where(u23, dd[3], dd[2])
        c23 = jnp.where(u23, jnp.int32(3), jnp.int32(2))
        u2 = m23 < m01
        mq = jnp.where(u2, m23, m01)
        cq = jnp.where(u2, c23, c01)
        upd = mq < m_t
        m_t = jnp.where(upd, mq, m_t)
        cix_t = jnp.where(upd, cq + jnp.int32(p * 4), cix_t)

    m = jnp.min(m_t, axis=1)
    giota = lax.broadcasted_iota(jnp.int32, (TM, TN), 1)
    gidx = cix_t * TN + giota
    ix = jnp.min(jnp.where(m_t == m[:, None], gidx, jnp.int32(2**30)), axis=1)

    idx_ref[0, 0, :] = ix
    part = jnp.sum(m)

    @pl.when(i == 0)
    def _():
        loss_ref[0, 0] = part

    @pl.when(i > 0)
    def _():
        loss_ref[0, 0] += part

    @pl.when(i == N_ROW_TILES - 1)
    def _():
        loss_ref[0, 0] = loss_ref[0, 0] * ((1.0 + BETA) / (B_TOT * E_DIM))


_argmin_call = pl.pallas_call(
    _argmin_body,
    grid=(N_ROW_TILES,),
    in_specs=[
        pl.BlockSpec((TM,), lambda i: (i,)),
        pl.BlockSpec((N_E,), lambda i: (0,)),
        pl.BlockSpec((TM, E_DIM), lambda i: (i, 0)),
        pl.BlockSpec((N_E, E_DIM), lambda i: (0, 0)),
    ],
    out_specs=[
        pl.BlockSpec((1, 1, TM), lambda i: (i, 0, 0)),
        pl.BlockSpec(memory_space=pltpu.SMEM),
    ],
    out_shape=[
        jax.ShapeDtypeStruct((N_ROW_TILES, 1, TM), jnp.int32),
        jax.ShapeDtypeStruct((1, 1), jnp.float32),
    ],
)

_NW = 32
_RPW = B_TOT // _NW
_CH = 3
_CW = _RPW // _CH
_HSL = N_E // 16

_sc_mesh = plsc.VectorSubcoreMesh(core_axis_name="c", subcore_axis_name="s")


@functools.partial(
    pl.kernel,
    out_type=[
        jax.ShapeDtypeStruct((B_TOT, E_DIM), jnp.float32),
        jax.ShapeDtypeStruct((2, N_E), jnp.float32),
    ],
    mesh=_sc_mesh,
    scratch_types=[
        pltpu.VMEM((_CH, _CW), jnp.int32),
        pltpu.VMEM((_RPW, E_DIM), jnp.float32),
        pltpu.VMEM((_CW,), jnp.float32),
        pltpu.VMEM((_HSL,), jnp.float32),
        pltpu.VMEM_SHARED((N_E,), jnp.float32),
        pltpu.SemaphoreType.DMA,
    ],
)
def _sc_gather_hist(emb_hbm, idx_hbm, zq_hbm, cnt_hbm,
                    idx_v, rows_v, ones_v, zer_v, hist_sh, sem):
    c = lax.axis_index("c")
    s = lax.axis_index("s")
    wid = s * 2 + c
    pltpu.sync_copy(idx_hbm.at[wid], idx_v)

    handles = [
        pltpu.async_copy(emb_hbm.at[idx_v.at[j]],
                         rows_v.at[pl.ds(j * _CW, _CW)], sem)
        for j in range(_CH)
    ]

    for k in range(_HSL // 16):
        zer_v[pl.ds(k * 16, 16)] = jnp.zeros((16,), jnp.float32)
    for k in range(_CW // 16):
        ones_v[pl.ds(k * 16, 16)] = jnp.ones((16,), jnp.float32)
    pltpu.sync_copy(zer_v, hist_sh.at[pl.ds(s * _HSL, _HSL)])
    plsc.subcore_barrier()

    for j in range(_CH):
        pltpu.sync_copy(ones_v, hist_sh.at[idx_v.at[j]], add=True)
    plsc.subcore_barrier()

    @pl.when(s == 0)
    def _():
        pltpu.sync_copy(hist_sh, cnt_hbm.at[c])

    for h in handles:
        h.wait()
    pltpu.sync_copy(rows_v, zq_hbm.at[pl.ds(wid * _RPW, _RPW)])


def _perp_body(cnt_ref, out_ref):
    cnt = cnt_ref[...]
    tot = cnt[0:8, :] + cnt[8:16, :]
    avg = tot / jnp.float32(B_TOT)
    ent = jnp.sum(avg * jnp.log(avg + 1e-12))
    out_ref[0, 0] = jnp.exp(-ent)


_perp_call = pl.pallas_call(
    _perp_body,
    in_specs=[pl.BlockSpec((16, 1024), lambda: (0, 0))],
    out_specs=pl.BlockSpec(memory_space=pltpu.SMEM),
    out_shape=jax.ShapeDtypeStruct((1, 1), jnp.float32),
)


def kernel(z, embedding):
    bz = z.shape[0]
    z_flat = z.reshape(-1, E_DIM)
    rn = jnp.sum(z_flat ** 2, axis=1)
    en = jnp.sum(embedding ** 2, axis=1)

    idx_tiles, loss = _argmin_call(rn, en, z_flat, embedding)
    idx_flat = idx_tiles.reshape(B_TOT)

    zq_flat, counts = _sc_gather_hist(embedding,
                                      idx_flat.reshape(_NW, _CH, _CW))
    perp = _perp_call(counts.reshape(16, 1024))

    z_q = zq_flat.reshape(bz, -1, E_DIM)
    return (z_q, loss.reshape(()), idx_flat, perp.reshape(()))


# speedup vs baseline: 2.3867x; 1.0400x over previous
import functools

import jax
import jax.numpy as jnp
from jax import lax
from jax.experimental import pallas as pl
from jax.experimental.pallas import tpu as pltpu
from jax.experimental.pallas import tpu_sc as plsc

N_E = 8192
E_DIM = 256
BETA = 0.25
B_TOT = 16 * 576

TM = 256
TN = 512
N_ROW_TILES = B_TOT // TM
N_COL_CHUNKS = N_E // TN


def _argmin_body(rn_ref, en_ref, z_ref, e_ref, idx_ref, loss_ref):
    i = pl.program_id(0)
    z_tile = z_ref[...]
    rn_tile = rn_ref[...]

    m_t = jnp.full((TM, TN), jnp.inf, jnp.float32)
    cix_t = jnp.zeros((TM, TN), jnp.int32)
    for p in range(N_COL_CHUNKS // 4):
        dd = []
        for q in range(4):
            c = p * 4 + q
            e_chunk = e_ref[pl.ds(c * TN, TN), :]
            en_chunk = en_ref[pl.ds(c * TN, TN)]
            mm2 = lax.dot_general(z_tile, e_chunk, (((1,), (1,)), ((), ())),
                                  preferred_element_type=jnp.float32)
            dd.append((rn_tile[:, None] + en_chunk[None, :]) - mm2)
        u01 = dd[1] < dd[0]
        m01 = jnp.where(u01, dd[1], dd[0])
        c01 = jnp.where(u01, jnp.int32(p * 4 + 1), jnp.int32(p * 4))
        u23 = dd[3] < dd[2]
        m23 = jnp.where(u23, dd[3], dd[2])
        c23 = jnp.where(u23, jnp.int32(p * 4 + 3), jnp.int32(p * 4 + 2))
        u2 = m23 < m01
        mq = jnp.where(u2, m23, m01)
        cq = jnp.where(u2, c23, c01)
        upd = mq < m_t
        m_t = jnp.where(upd, mq, m_t)
        cix_t = jnp.where(upd, cq, cix_t)

    m = jnp.min(m_t, axis=1)
    giota = lax.broadcasted_iota(jnp.int32, (TM, TN), 1)
    gidx = cix_t * TN + giota
    ix = jnp.min(jnp.where(m_t == m[:, None], gidx, jnp.int32(2**30)), axis=1)

    idx_ref[0, 0, :] = ix
    part = jnp.sum(m)

    @pl.when(i == 0)
    def _():
        loss_ref[0, 0] = part

    @pl.when(i > 0)
    def _():
        loss_ref[0, 0] += part

    @pl.when(i == N_ROW_TILES - 1)
    def _():
        loss_ref[0, 0] = loss_ref[0, 0] * ((1.0 + BETA) / (B_TOT * E_DIM))


_argmin_call = pl.pallas_call(
    _argmin_body,
    grid=(N_ROW_TILES,),
    in_specs=[
        pl.BlockSpec((TM,), lambda i: (i,)),
        pl.BlockSpec((N_E,), lambda i: (0,)),
        pl.BlockSpec((TM, E_DIM), lambda i: (i, 0)),
        pl.BlockSpec((N_E, E_DIM), lambda i: (0, 0)),
    ],
    out_specs=[
        pl.BlockSpec((1, 1, TM), lambda i: (i, 0, 0)),
        pl.BlockSpec(memory_space=pltpu.SMEM),
    ],
    out_shape=[
        jax.ShapeDtypeStruct((N_ROW_TILES, 1, TM), jnp.int32),
        jax.ShapeDtypeStruct((1, 1), jnp.float32),
    ],
)

_NW = 32
_RPW = B_TOT // _NW
_CH = 3
_CW = _RPW // _CH
_HSL = N_E // 16

_sc_mesh = plsc.VectorSubcoreMesh(core_axis_name="c", subcore_axis_name="s")


@functools.partial(
    pl.kernel,
    out_type=[
        jax.ShapeDtypeStruct((B_TOT, E_DIM), jnp.float32),
        jax.ShapeDtypeStruct((2, N_E), jnp.float32),
    ],
    mesh=_sc_mesh,
    scratch_types=[
        pltpu.VMEM((_CH, _CW), jnp.int32),
        pltpu.VMEM((_RPW, E_DIM), jnp.float32),
        pltpu.VMEM((_CW,), jnp.float32),
        pltpu.VMEM((_HSL,), jnp.float32),
        pltpu.VMEM_SHARED((N_E,), jnp.float32),
        pltpu.SemaphoreType.DMA,
    ],
)
def _sc_gather_hist(emb_hbm, idx_hbm, zq_hbm, cnt_hbm,
                    idx_v, rows_v, ones_v, zer_v, hist_sh, sem):
    c = lax.axis_index("c")
    s = lax.axis_index("s")
    wid = s * 2 + c
    pltpu.sync_copy(idx_hbm.at[wid], idx_v)

    handles = [
        pltpu.async_copy(emb_hbm.at[idx_v.at[j]],
                         rows_v.at[pl.ds(j * _CW, _CW)], sem)
        for j in range(_CH)
    ]

    for k in range(_HSL // 16):
        zer_v[pl.ds(k * 16, 16)] = jnp.zeros((16,), jnp.float32)
    for k in range(_CW // 16):
        ones_v[pl.ds(k * 16, 16)] = jnp.ones((16,), jnp.float32)
    pltpu.sync_copy(zer_v, hist_sh.at[pl.ds(s * _HSL, _HSL)])
    plsc.subcore_barrier()

    for j in range(_CH):
        pltpu.sync_copy(ones_v, hist_sh.at[idx_v.at[j]], add=True)
    plsc.subcore_barrier()

    @pl.when(s == 0)
    def _():
        pltpu.sync_copy(hist_sh, cnt_hbm.at[c])

    for h in handles:
        h.wait()
    pltpu.sync_copy(rows_v, zq_hbm.at[pl.ds(wid * _RPW, _RPW)])


def _perp_body(cnt_ref, out_ref):
    cnt = cnt_ref[...]
    tot = cnt[0:8, :] + cnt[8:16, :]
    avg = tot / jnp.float32(B_TOT)
    ent = jnp.sum(avg * jnp.log(avg + 1e-12))
    out_ref[0, 0] = jnp.exp(-ent)


_perp_call = pl.pallas_call(
    _perp_body,
    in_specs=[pl.BlockSpec((16, 1024), lambda: (0, 0))],
    out_specs=pl.BlockSpec(memory_space=pltpu.SMEM),
    out_shape=jax.ShapeDtypeStruct((1, 1), jnp.float32),
)


def kernel(z, embedding):
    bz = z.shape[0]
    z_flat = z.reshape(-1, E_DIM)
    rn = jnp.sum(z_flat ** 2, axis=1)
    en = jnp.sum(embedding ** 2, axis=1)

    idx_tiles, loss = _argmin_call(rn, en, z_flat, embedding * 2.0)
    idx_flat = idx_tiles.reshape(B_TOT)

    zq_flat, counts = _sc_gather_hist(embedding,
                                      idx_flat.reshape(_NW, _CH, _CW))
    perp = _perp_call(counts.reshape(16, 1024))

    z_q = zq_flat.reshape(bz, -1, E_DIM)
    return (z_q, loss.reshape(()), idx_flat, perp.reshape(()))


# speedup vs baseline: 2.4098x; 1.0097x over previous
import functools

import jax
import jax.numpy as jnp
from jax import lax
from jax.experimental import pallas as pl
from jax.experimental.pallas import tpu as pltpu
from jax.experimental.pallas import tpu_sc as plsc

N_E = 8192
E_DIM = 256
BETA = 0.25
B_TOT = 16 * 576

TM = 256
TN = 512
N_ROW_TILES = B_TOT // TM
N_COL_CHUNKS = N_E // TN


def _argmin_body(rn_ref, en_ref, z_ref, e_ref, idx_ref, loss_ref):
    i = pl.program_id(0)
    z_tile = z_ref[...]
    rn_tile = rn_ref[...]

    STRIPS = 4
    SW = TN // STRIPS
    m_st = [jnp.full((TM, SW), jnp.inf, jnp.float32) for _ in range(STRIPS)]
    c_st = [jnp.zeros((TM, SW), jnp.int32) for _ in range(STRIPS)]
    rn_col = rn_tile[:, None]
    for p in range(N_COL_CHUNKS // 4):
        mms = []
        ens = []
        for q in range(4):
            c = p * 4 + q
            e_chunk = e_ref[pl.ds(c * TN, TN), :]
            ens.append(en_ref[pl.ds(c * TN, TN)])
            mms.append(lax.dot_general(z_tile, e_chunk,
                                       (((1,), (1,)), ((), ())),
                                       preferred_element_type=jnp.float32))
        for s in range(STRIPS):
            lo = s * SW
            d0 = (rn_col + ens[0][lo:lo + SW][None, :]) - mms[0][:, lo:lo + SW]
            d1 = (rn_col + ens[1][lo:lo + SW][None, :]) - mms[1][:, lo:lo + SW]
            u01 = d1 < d0
            m01 = jnp.where(u01, d1, d0)
            c01 = jnp.where(u01, jnp.int32(p * 4 + 1), jnp.int32(p * 4))
            d2 = (rn_col + ens[2][lo:lo + SW][None, :]) - mms[2][:, lo:lo + SW]
            d3 = (rn_col + ens[3][lo:lo + SW][None, :]) - mms[3][:, lo:lo + SW]
            u23 = d3 < d2
            m23 = jnp.where(u23, d3, d2)
            c23 = jnp.where(u23, jnp.int32(p * 4 + 3), jnp.int32(p * 4 + 2))
            u2 = m23 < m01
            mq = jnp.where(u2, m23, m01)
            cq = jnp.where(u2, c23, c01)
            upd = mq < m_st[s]
            m_st[s] = jnp.where(upd, mq, m_st[s])
            c_st[s] = jnp.where(upd, cq, c_st[s])

    mrow = jnp.minimum(jnp.minimum(m_st[0], m_st[1]),
                       jnp.minimum(m_st[2], m_st[3]))
    m = jnp.min(mrow, axis=1)
    siota = lax.broadcasted_iota(jnp.int32, (TM, SW), 1)
    BIG = jnp.int32(2**30)
    cand = jnp.full((TM, SW), BIG, jnp.int32)
    for s in range(STRIPS):
        gidx = c_st[s] * TN + (siota + jnp.int32(s * SW))
        cand = jnp.minimum(cand,
                           jnp.where(m_st[s] == m[:, None], gidx, BIG))
    ix = jnp.min(cand, axis=1)

    idx_ref[0, 0, :] = ix
    part = jnp.sum(m)

    @pl.when(i == 0)
    def _():
        loss_ref[0, 0] = part

    @pl.when(i > 0)
    def _():
        loss_ref[0, 0] += part

    @pl.when(i == N_ROW_TILES - 1)
    def _():
        loss_ref[0, 0] = loss_ref[0, 0] * ((1.0 + BETA) / (B_TOT * E_DIM))


_argmin_call = pl.pallas_call(
    _argmin_body,
    grid=(N_ROW_TILES,),
    in_specs=[
        pl.BlockSpec((TM,), lambda i: (i,)),
        pl.BlockSpec((N_E,), lambda i: (0,)),
        pl.BlockSpec((TM, E_DIM), lambda i: (i, 0)),
        pl.BlockSpec((N_E, E_DIM), lambda i: (0, 0)),
    ],
    out_specs=[
        pl.BlockSpec((1, 1, TM), lambda i: (i, 0, 0)),
        pl.BlockSpec(memory_space=pltpu.SMEM),
    ],
    out_shape=[
        jax.ShapeDtypeStruct((N_ROW_TILES, 1, TM), jnp.int32),
        jax.ShapeDtypeStruct((1, 1), jnp.float32),
    ],
)

_NW = 32
_RPW = B_TOT // _NW
_CH = 3
_CW = _RPW // _CH
_HSL = N_E // 16

_sc_mesh = plsc.VectorSubcoreMesh(core_axis_name="c", subcore_axis_name="s")


@functools.partial(
    pl.kernel,
    out_type=[
        jax.ShapeDtypeStruct((B_TOT, E_DIM), jnp.float32),
        jax.ShapeDtypeStruct((2, N_E), jnp.float32),
    ],
    mesh=_sc_mesh,
    scratch_types=[
        pltpu.VMEM((_CH, _CW), jnp.int32),
        pltpu.VMEM((_RPW, E_DIM), jnp.float32),
        pltpu.VMEM((_CW,), jnp.float32),
        pltpu.VMEM((_HSL,), jnp.float32),
        pltpu.VMEM_SHARED((N_E,), jnp.float32),
        pltpu.SemaphoreType.DMA,
    ],
)
def _sc_gather_hist(emb_hbm, idx_hbm, zq_hbm, cnt_hbm,
                    idx_v, rows_v, ones_v, zer_v, hist_sh, sem):
    c = lax.axis_index("c")
    s = lax.axis_index("s")
    wid = s * 2 + c
    pltpu.sync_copy(idx_hbm.at[wid], idx_v)

    handles = [
        pltpu.async_copy(emb_hbm.at[idx_v.at[j]],
                         rows_v.at[pl.ds(j * _CW, _CW)], sem)
        for j in range(_CH)
    ]

    for k in range(_HSL // 16):
        zer_v[pl.ds(k * 16, 16)] = jnp.zeros((16,), jnp.float32)
    for k in range(_CW // 16):
        ones_v[pl.ds(k * 16, 16)] = jnp.ones((16,), jnp.float32)
    pltpu.sync_copy(zer_v, hist_sh.at[pl.ds(s * _HSL, _HSL)])
    plsc.subcore_barrier()

    for j in range(_CH):
        pltpu.sync_copy(ones_v, hist_sh.at[idx_v.at[j]], add=True)
    plsc.subcore_barrier()

    @pl.when(s == 0)
    def _():
        pltpu.sync_copy(hist_sh, cnt_hbm.at[c])

    for h in handles:
        h.wait()
    pltpu.sync_copy(rows_v, zq_hbm.at[pl.ds(wid * _RPW, _RPW)])


def _perp_body(cnt_ref, out_ref):
    cnt = cnt_ref[...]
    tot = cnt[0:8, :] + cnt[8:16, :]
    avg = tot / jnp.float32(B_TOT)
    ent = jnp.sum(avg * jnp.log(avg + 1e-12))
    out_ref[0, 0] = jnp.exp(-ent)


_perp_call = pl.pallas_call(
    _perp_body,
    in_specs=[pl.BlockSpec((16, 1024), lambda: (0, 0))],
    out_specs=pl.BlockSpec(memory_space=pltpu.SMEM),
    out_shape=jax.ShapeDtypeStruct((1, 1), jnp.float32),
)


def kernel(z, embedding):
    bz = z.shape[0]
    z_flat = z.reshape(-1, E_DIM)
    rn = jnp.sum(z_flat ** 2, axis=1)
    en = jnp.sum(embedding ** 2, axis=1)

    idx_tiles, loss = _argmin_call(rn, en, z_flat, embedding * 2.0)
    idx_flat = idx_tiles.reshape(B_TOT)

    zq_flat, counts = _sc_gather_hist(embedding,
                                      idx_flat.reshape(_NW, _CH, _CW))
    perp = _perp_call(counts.reshape(16, 1024))

    z_q = zq_flat.reshape(bz, -1, E_DIM)
    return (z_q, loss.reshape(()), idx_flat, perp.reshape(()))
